# baseline (speedup 1.0000x reference)
import functools

import jax
import jax.numpy as jnp
from jax import lax
from jax.experimental import pallas as pl
from jax.experimental.pallas import tpu as pltpu
from jax.experimental.pallas import tpu_sc as plsc

KEEP_FRAC = 0.25
_ROWS = 2048
_ROWS_MAP = 1024
_CHUNK = 256
_SC_ROWS = 6144
_NW = 32
_CH = 16
_LANES = 16


def _mask_sumsq_kernel(imp_row_ref, imp_col_ref, x_ref,
                       mask_ref, sumsq_ref, acc_ref, mask_vmem):
    step = pl.program_id(0)
    nsteps = pl.num_programs(0)
    d = imp_row_ref.shape[1]
    k = max(1, int(d * KEEP_FRAC))

    @pl.when(step == 0)
    def _():
        iota_j = jax.lax.broadcasted_iota(jnp.int32, (1, d), 1)
        iota_chunk = jax.lax.broadcasted_iota(jnp.int32, (_CHUNK, 1), 0)
        imp_row = imp_row_ref[...]

        def body(c, rank):
            vi = imp_col_ref[pl.ds(c * _CHUNK, _CHUNK), :]
            ii = iota_chunk + c * _CHUNK
            beat = (vi > imp_row) | ((vi == imp_row) & (ii < iota_j))
            return rank + jnp.sum(beat.astype(jnp.int32), axis=0, keepdims=True)

        rank = jax.lax.fori_loop(0, d // _CHUNK, body,
                                 jnp.zeros((1, d), jnp.int32))
        m = (rank < k).astype(jnp.float32)
        mask_vmem[...] = m
        mask_ref[...] = m
        acc_ref[0] = 0.0

    xb = x_ref[...]
    unsel = 1.0 - mask_vmem[...]
    acc_ref[0] += jnp.sum(xb * xb * unsel)

    @pl.when(step == nsteps - 1)
    def _():
        sumsq_ref[0] = acc_ref[0]


def _make_sc_colsq(d, start, rows_pw):
    cols_vregs = d // _LANES

    def body(x_ref, out_ref, buf0, buf1, acc, sem0, sem1):
        nc = 2
        wid = lax.axis_index("s") * nc + lax.axis_index("c")
        base_row = start + wid * rows_pw
        n_chunks = rows_pw // _CH

        for j in range(cols_vregs):
            acc[pl.ds(j * _LANES, _LANES)] = jnp.zeros((_LANES,), jnp.float32)

        def issue(g):
            buf = buf0 if g % 2 == 0 else buf1
            sem = sem0 if g % 2 == 0 else sem1
            return pltpu.async_copy(
                x_ref.at[pl.ds(base_row + g * _CH, _CH)], buf, sem)

        cp = issue(0)
        for g in range(n_chunks):
            nxt = issue(g + 1) if g + 1 < n_chunks else None
            cp.wait()
            buf = buf0 if g % 2 == 0 else buf1

            @plsc.parallel_loop(0, cols_vregs, unroll=4)
            def _(c):
                col = c * _LANES
                s = None
                for r in range(_CH):
                    v = buf[r, pl.ds(col, _LANES)]
                    s = v * v if s is None else s + v * v
                plsc.addupdate(acc.at[pl.ds(col, _LANES)], s)

            cp = nxt

        pltpu.sync_copy(acc, out_ref.at[wid])

    return pl.kernel(
        body,
        out_type=jax.ShapeDtypeStruct((_NW, d), jnp.float32),
        mesh=plsc.VectorSubcoreMesh(core_axis_name="c", subcore_axis_name="s"),
        scratch_types=[
            pltpu.VMEM((_CH, d), jnp.float32),
            pltpu.VMEM((_CH, d), jnp.float32),
            pltpu.VMEM((d,), jnp.float32),
            pltpu.SemaphoreType.DMA,
            pltpu.SemaphoreType.DMA,
        ],
    )


def _apply_kernel(sumsq_ref, params_ref, mask_ref, colsq_ref, x_ref, y_ref,
                  loss_ref):
    step = pl.program_id(0)

    @pl.when(step == 0)
    def _():
        unsel = 1.0 - mask_ref[...]
        sc_part = jnp.sum(colsq_ref[...] * unsel)
        loss_ref[0] = jnp.sqrt(sumsq_ref[0] + sc_part) * 1e-6

    loss = loss_ref[0]
    w0 = params_ref[0]
    b0 = params_ref[1]
    w1 = params_ref[2]
    b1 = params_ref[3]
    xb = x_ref[...]
    t = xb * w0 + b0
    t = t * jax.lax.logistic(t)
    t = t * w1 + b1
    t = t * jax.lax.logistic(t)
    m = mask_ref[...]
    y_ref[...] = xb + m * (t - xb) + loss


def kernel(x, importance, w0, b0, w1, b1):
    b_, t_, d_ = x.shape
    n = b_ * t_
    x2 = x.reshape(n, d_)
    imp_row = importance.reshape(1, d_)
    imp_col = importance.reshape(d_, 1)
    params = jnp.concatenate(
        [w0.reshape(-1), b0.reshape(-1), w1.reshape(-1), b1.reshape(-1)])

    sc_rows = _SC_ROWS
    tc_rows = n - sc_rows
    nb1 = tc_rows // _ROWS

    sc_colsq = _make_sc_colsq(d_, tc_rows, sc_rows // _NW)(x2)

    mask, sumsq = pl.pallas_call(
        _mask_sumsq_kernel,
        grid=(nb1,),
        in_specs=[
            pl.BlockSpec((1, d_), lambda b: (0, 0)),
            pl.BlockSpec((d_, 1), lambda b: (0, 0)),
            pl.BlockSpec((_ROWS, d_), lambda b: (b, 0)),
        ],
        out_specs=[
            pl.BlockSpec((1, d_), lambda b: (0, 0)),
            pl.BlockSpec(memory_space=pltpu.SMEM),
        ],
        out_shape=[
            jax.ShapeDtypeStruct((1, d_), jnp.float32),
            jax.ShapeDtypeStruct((1,), jnp.float32),
        ],
        scratch_shapes=[
            pltpu.SMEM((1,), jnp.float32),
            pltpu.VMEM((1, d_), jnp.float32),
        ],
    )(imp_row, imp_col, x2)

    nb2 = n // _ROWS_MAP
    y2 = pl.pallas_call(
        _apply_kernel,
        grid=(nb2,),
        in_specs=[
            pl.BlockSpec(memory_space=pltpu.SMEM),
            pl.BlockSpec(memory_space=pltpu.SMEM),
            pl.BlockSpec((1, d_), lambda b: (0, 0)),
            pl.BlockSpec((_NW, d_), lambda b: (0, 0)),
            pl.BlockSpec((_ROWS_MAP, d_), lambda b: (b, 0)),
        ],
        out_specs=pl.BlockSpec((_ROWS_MAP, d_), lambda b: (b, 0)),
        out_shape=jax.ShapeDtypeStruct((n, d_), jnp.float32),
        scratch_shapes=[pltpu.SMEM((1,), jnp.float32)],
    )(sumsq, params, mask, sc_colsq, x2)

    return y2.reshape(b_, t_, d_)


# speedup vs baseline: 1.0090x; 1.0090x over previous
import functools

import jax
import jax.numpy as jnp
from jax import lax
from jax.experimental import pallas as pl
from jax.experimental.pallas import tpu as pltpu
from jax.experimental.pallas import tpu_sc as plsc

KEEP_FRAC = 0.25
_ROWS = 2048
_ROWS_MAP = 1024
_CHUNK = 256
_SC_ROWS = 8192
_NW = 32
_CH = 16
_LANES = 16


def _mask_sumsq_kernel(imp_row_ref, imp_col_ref, x_ref,
                       mask_ref, sumsq_ref, acc_ref, mask_vmem):
    step = pl.program_id(0)
    nsteps = pl.num_programs(0)
    d = imp_row_ref.shape[1]
    k = max(1, int(d * KEEP_FRAC))

    @pl.when(step == 0)
    def _():
        iota_j = jax.lax.broadcasted_iota(jnp.int32, (1, d), 1)
        iota_chunk = jax.lax.broadcasted_iota(jnp.int32, (_CHUNK, 1), 0)
        imp_row = imp_row_ref[...]

        def body(c, rank):
            vi = imp_col_ref[pl.ds(c * _CHUNK, _CHUNK), :]
            ii = iota_chunk + c * _CHUNK
            beat = (vi > imp_row) | ((vi == imp_row) & (ii < iota_j))
            return rank + jnp.sum(beat.astype(jnp.int32), axis=0, keepdims=True)

        rank = jax.lax.fori_loop(0, d // _CHUNK, body,
                                 jnp.zeros((1, d), jnp.int32))
        m = (rank < k).astype(jnp.float32)
        mask_vmem[...] = m
        mask_ref[...] = m
        acc_ref[0] = 0.0

    xb = x_ref[...]
    unsel = 1.0 - mask_vmem[...]
    acc_ref[0] += jnp.sum(xb * xb * unsel)

    @pl.when(step == nsteps - 1)
    def _():
        sumsq_ref[0] = acc_ref[0]


def _make_sc_colsq(d, start, rows_pw):
    cols_vregs = d // _LANES

    def body(x_ref, out_ref, buf0, buf1, buf2, acc, sem0, sem1, sem2):
        nc = 2
        wid = lax.axis_index("s") * nc + lax.axis_index("c")
        base_row = start + wid * rows_pw
        n_chunks = rows_pw // _CH

        for j in range(cols_vregs):
            acc[pl.ds(j * _LANES, _LANES)] = jnp.zeros((_LANES,), jnp.float32)

        bufs = (buf0, buf1, buf2)
        sems = (sem0, sem1, sem2)

        def issue(g):
            return pltpu.async_copy(
                x_ref.at[pl.ds(base_row + g * _CH, _CH)],
                bufs[g % 3], sems[g % 3])

        cps = {g: issue(g) for g in range(min(2, n_chunks))}
        for g in range(n_chunks):
            if g + 2 < n_chunks:
                cps[g + 2] = issue(g + 2)
            cps[g].wait()
            buf = bufs[g % 3]

            @plsc.parallel_loop(0, cols_vregs, unroll=4)
            def _(c):
                col = c * _LANES
                s = None
                for r in range(_CH):
                    v = buf[r, pl.ds(col, _LANES)]
                    s = v * v if s is None else s + v * v
                plsc.addupdate(acc.at[pl.ds(col, _LANES)], s)

        pltpu.sync_copy(acc, out_ref.at[wid])

    return pl.kernel(
        body,
        out_type=jax.ShapeDtypeStruct((_NW, d), jnp.float32),
        mesh=plsc.VectorSubcoreMesh(core_axis_name="c", subcore_axis_name="s"),
        scratch_types=[
            pltpu.VMEM((_CH, d), jnp.float32),
            pltpu.VMEM((_CH, d), jnp.float32),
            pltpu.VMEM((_CH, d), jnp.float32),
            pltpu.VMEM((d,), jnp.float32),
            pltpu.SemaphoreType.DMA,
            pltpu.SemaphoreType.DMA,
            pltpu.SemaphoreType.DMA,
        ],
    )


def _apply_kernel(sumsq_ref, params_ref, mask_ref, colsq_ref, x_ref, y_ref,
                  loss_ref):
    step = pl.program_id(0)

    @pl.when(step == 0)
    def _():
        unsel = 1.0 - mask_ref[...]
        sc_part = jnp.sum(colsq_ref[...] * unsel)
        loss_ref[0] = jnp.sqrt(sumsq_ref[0] + sc_part) * 1e-6

    loss = loss_ref[0]
    w0 = params_ref[0]
    b0 = params_ref[1]
    w1 = params_ref[2]
    b1 = params_ref[3]
    xb = x_ref[...]
    t = xb * w0 + b0
    t = t * jax.lax.logistic(t)
    t = t * w1 + b1
    t = t * jax.lax.logistic(t)
    m = mask_ref[...]
    y_ref[...] = xb + m * (t - xb) + loss


def kernel(x, importance, w0, b0, w1, b1):
    b_, t_, d_ = x.shape
    n = b_ * t_
    x2 = x.reshape(n, d_)
    imp_row = importance.reshape(1, d_)
    imp_col = importance.reshape(d_, 1)
    params = jnp.concatenate(
        [w0.reshape(-1), b0.reshape(-1), w1.reshape(-1), b1.reshape(-1)])

    sc_rows = _SC_ROWS
    tc_rows = n - sc_rows
    nb1 = tc_rows // _ROWS

    sc_colsq = _make_sc_colsq(d_, tc_rows, sc_rows // _NW)(x2)

    mask, sumsq = pl.pallas_call(
        _mask_sumsq_kernel,
        grid=(nb1,),
        in_specs=[
            pl.BlockSpec((1, d_), lambda b: (0, 0)),
            pl.BlockSpec((d_, 1), lambda b: (0, 0)),
            pl.BlockSpec((_ROWS, d_), lambda b: (b, 0)),
        ],
        out_specs=[
            pl.BlockSpec((1, d_), lambda b: (0, 0)),
            pl.BlockSpec(memory_space=pltpu.SMEM),
        ],
        out_shape=[
            jax.ShapeDtypeStruct((1, d_), jnp.float32),
            jax.ShapeDtypeStruct((1,), jnp.float32),
        ],
        scratch_shapes=[
            pltpu.SMEM((1,), jnp.float32),
            pltpu.VMEM((1, d_), jnp.float32),
        ],
    )(imp_row, imp_col, x2)

    nb2 = n // _ROWS_MAP
    y2 = pl.pallas_call(
        _apply_kernel,
        grid=(nb2,),
        in_specs=[
            pl.BlockSpec(memory_space=pltpu.SMEM),
            pl.BlockSpec(memory_space=pltpu.SMEM),
            pl.BlockSpec((1, d_), lambda b: (0, 0)),
            pl.BlockSpec((_NW, d_), lambda b: (0, 0)),
            pl.BlockSpec((_ROWS_MAP, d_), lambda b: (b, 0)),
        ],
        out_specs=pl.BlockSpec((_ROWS_MAP, d_), lambda b: (b, 0)),
        out_shape=jax.ShapeDtypeStruct((n, d_), jnp.float32),
        scratch_shapes=[pltpu.SMEM((1,), jnp.float32)],
    )(sumsq, params, mask, sc_colsq, x2)

    return y2.reshape(b_, t_, d_)
